# baseline (speedup 1.0000x reference)
import functools

import jax
import jax.numpy as jnp
from jax import lax
from jax.experimental import pallas as pl
from jax.experimental.pallas import tpu as pltpu
from jax.experimental.pallas import tpu_sc as plsc

N = 10000
E = 320000
H = 128
NC = 2
NS = 16
NT = NC * NS
NP = 10240
EPT = E // NT
CH = 100
NCHT = EPT // CH
RB = 400
GRID = N // RB


@functools.cache
def _sc_kernels():
    mesh = plsc.VectorSubcoreMesh(core_axis_name="c", subcore_axis_name="s",
                                  num_cores=NC, num_subcores=NS)

    @functools.partial(
        pl.kernel,
        out_type=jax.ShapeDtypeStruct((NT, NP), jnp.float32),
        mesh=mesh,
        compiler_params=pltpu.CompilerParams(needs_layout_passes=False),
        scratch_types=[
            pltpu.VMEM((EPT,), jnp.int32),
            pltpu.VMEM((NP,), jnp.float32),
        ],
    )
    def _sc_degree(dst_hbm, out_hbm, dst_v, hist_v):
        cid = lax.axis_index("c")
        sid = lax.axis_index("s")
        wid = cid * NS + sid
        zero = jnp.zeros((16,), jnp.float32)

        def zbody(i, carry):
            hist_v[pl.ds(i * 16, 16)] = zero
            return carry

        lax.fori_loop(0, NP // 16, zbody, 0)
        pltpu.sync_copy(dst_hbm.at[pl.ds(wid * EPT, EPT)], dst_v)
        ones = jnp.ones((16,), jnp.float32)

        def body(i, carry):
            idx = dst_v[pl.ds(i * 16, 16)]
            plsc.addupdate_scatter(hist_v, [idx], ones)
            return carry

        lax.fori_loop(0, EPT // 16, body, 0)
        pltpu.sync_copy(hist_v, out_hbm.at[wid])

    @functools.partial(
        pl.kernel,
        out_type=jax.ShapeDtypeStruct((NC, NP, H), jnp.float32),
        mesh=mesh,
        compiler_params=pltpu.CompilerParams(needs_layout_passes=False,
                                             use_tc_tiling_on_sc=False),
        scratch_types=[
            pltpu.VMEM((NCHT, CH), jnp.int32),
            pltpu.VMEM((NCHT, CH), jnp.int32),
            pltpu.VMEM((CH, H), jnp.float32),
            pltpu.VMEM((CH, H), jnp.float32),
            pltpu.VMEM_SHARED((NP, H), jnp.float32),
            pltpu.SemaphoreType.DMA,
            pltpu.SemaphoreType.DMA,
        ],
    )
    def _sc_aggregate(hp_hbm, ei_hbm, out_hbm,
                      sidx_v, didx_v, rows0_v, rows1_v, acc_sh,
                      semA, semB):
        cid = lax.axis_index("c")
        sid = lax.axis_index("s")
        wid = cid * NS + sid
        zero = jnp.zeros((16,), jnp.float32)

        def zbody(i, carry):
            rows0_v[i >> 3, pl.ds((i & 7) * 16, 16)] = zero
            return carry

        lax.fori_loop(0, CH * (H // 16), zbody, 0)
        base = sid * (NP // NS)
        _nz, _rz = (NP // NS) // CH, (NP // NS) % CH
        for j in range(_nz):
            pltpu.sync_copy(rows0_v, acc_sh.at[pl.ds(base + j * CH, CH)])
        if _rz:
            pltpu.sync_copy(rows0_v.at[pl.ds(0, _rz)],
                            acc_sh.at[pl.ds(base + _nz * CH, _rz)])
        plsc.subcore_barrier()

        pltpu.sync_copy(ei_hbm.at[0, pl.ds(wid * NCHT, NCHT)], sidx_v)
        pltpu.sync_copy(ei_hbm.at[1, pl.ds(wid * NCHT, NCHT)], didx_v)

        def _wait(buf, sem):
            pltpu.make_async_copy(hp_hbm.at[pl.ds(0, CH)], buf, sem).wait()

        pltpu.async_copy(hp_hbm.at[sidx_v.at[0]], rows0_v, semA)

        def body(i, carry):
            j = 2 * i
            pltpu.async_copy(hp_hbm.at[sidx_v.at[j + 1]], rows1_v, semB)
            _wait(rows0_v, semA)
            pltpu.sync_copy(rows0_v, acc_sh.at[didx_v.at[j]], add=True)

            @pl.when(j + 2 < NCHT)
            def _():
                pltpu.async_copy(hp_hbm.at[sidx_v.at[j + 2]], rows0_v, semA)

            _wait(rows1_v, semB)
            pltpu.sync_copy(rows1_v, acc_sh.at[didx_v.at[j + 1]], add=True)
            return carry

        lax.fori_loop(0, NCHT // 2, body, 0)
        plsc.subcore_barrier()
        for j in range(NP // NS // 128):
            pltpu.sync_copy(acc_sh.at[pl.ds(base + j * 128, 128)],
                            out_hbm.at[cid, pl.ds(base + j * 128, 128)])

    return _sc_degree, _sc_aggregate


def _dot(a, b):
    return jnp.dot(a, b, preferred_element_type=jnp.float32,
                   precision=jax.lax.Precision.HIGHEST)


def _tc_first_body(x_ref, w_ref, degT_ref, hp_ref, dinv_ref):
    deg = jnp.sum(degT_ref[...], axis=1, keepdims=True) + 1.0
    dinv = lax.rsqrt(deg)
    h = _dot(x_ref[...], w_ref[...])
    hp_ref[...] = h * dinv
    dinv_ref[...] = dinv


def _tc_first(x, W, degT):
    return pl.pallas_call(
        _tc_first_body,
        grid=(GRID,),
        in_specs=[pl.BlockSpec((RB, H), lambda i: (i, 0)),
                  pl.BlockSpec((H, H), lambda i: (0, 0)),
                  pl.BlockSpec((RB, NT), lambda i: (i, 0))],
        out_specs=[pl.BlockSpec((RB, H), lambda i: (i, 0)),
                   pl.BlockSpec((RB, 1), lambda i: (i, 0))],
        out_shape=[jax.ShapeDtypeStruct((N, H), jnp.float32),
                   jax.ShapeDtypeStruct((N, 1), jnp.float32)],
    )(x, W, degT)


def _bn_coeffs(stats, g, be):
    m = stats[0:1, :] * (1.0 / N)
    v = stats[1:2, :] * (1.0 / N) - m * m
    a = g * lax.rsqrt(v + 1e-5)
    return a, be - m * a


def _post_phase(agg_ref, hp_ref, dinv_ref, b_ref, zs_ref, stats_ref, i):
    s = agg_ref[0] + agg_ref[1] + hp_ref[...]
    z = jnp.maximum(s * dinv_ref[...] + b_ref[...], 0.0)
    zs_ref[pl.ds(i * RB, RB), :] = z
    new = jnp.concatenate([jnp.sum(z, axis=0, keepdims=True),
                           jnp.sum(z * z, axis=0, keepdims=True)], axis=0)

    @pl.when(i == 0)
    def _():
        stats_ref[...] = new

    @pl.when(i > 0)
    def _():
        stats_ref[...] += new


def _tc_postmm_body(agg_ref, hp_ref, dinv_ref, b_ref, g_ref, be_ref, w_ref,
                    out_ref, zs_ref, stats_ref):
    p = pl.program_id(0)
    i = pl.program_id(1)

    @pl.when(p == 0)
    def _():
        _post_phase(agg_ref, hp_ref, dinv_ref, b_ref, zs_ref, stats_ref, i)

    @pl.when(p == 1)
    def _():
        a, c = _bn_coeffs(stats_ref[...], g_ref[...], be_ref[...])
        z = zs_ref[pl.ds(i * RB, RB), :]
        h = _dot(z * a, w_ref[...]) + _dot(c, w_ref[...])
        out_ref[...] = h * dinv_ref[...]


def _tc_postmm(agg, hp, dinv, b, g, be, W):
    return pl.pallas_call(
        _tc_postmm_body,
        grid=(2, GRID),
        in_specs=[pl.BlockSpec((NC, RB, H), lambda p, i: (0, i, 0)),
                  pl.BlockSpec((RB, H), lambda p, i: (i, 0)),
                  pl.BlockSpec((RB, 1), lambda p, i: (i, 0)),
                  pl.BlockSpec((1, H), lambda p, i: (0, 0)),
                  pl.BlockSpec((1, H), lambda p, i: (0, 0)),
                  pl.BlockSpec((1, H), lambda p, i: (0, 0)),
                  pl.BlockSpec((H, H), lambda p, i: (0, 0))],
        out_specs=pl.BlockSpec((RB, H), lambda p, i: (p * i, 0)),
        out_shape=jax.ShapeDtypeStruct((N, H), jnp.float32),
        scratch_shapes=[pltpu.VMEM((N, H), jnp.float32),
                        pltpu.VMEM((2, H), jnp.float32)],
    )(agg, hp, dinv, b, g, be, W)


def _tc_postfinal_body(agg_ref, hp_ref, dinv_ref, b_ref, g_ref, be_ref,
                       w_ref, bl_ref, out_ref, zs_ref, stats_ref):
    p = pl.program_id(0)
    i = pl.program_id(1)

    @pl.when(p == 0)
    def _():
        _post_phase(agg_ref, hp_ref, dinv_ref, b_ref, zs_ref, stats_ref, i)

    @pl.when(p == 1)
    def _():
        a, c = _bn_coeffs(stats_ref[...], g_ref[...], be_ref[...])
        z = zs_ref[pl.ds(i * RB, RB), :]
        t = _dot(z * a, w_ref[...]) + _dot(c, w_ref[...]) + bl_ref[...]
        r = jnp.maximum(t, 0.0)
        e = jnp.exp(r - jnp.max(r, axis=1, keepdims=True))
        out_ref[...] = e / jnp.sum(e, axis=1, keepdims=True)


def _tc_postfinal(agg, hp, dinv, b, g, be, W, bl):
    return pl.pallas_call(
        _tc_postfinal_body,
        grid=(2, GRID),
        in_specs=[pl.BlockSpec((NC, RB, H), lambda p, i: (0, i, 0)),
                  pl.BlockSpec((RB, H), lambda p, i: (i, 0)),
                  pl.BlockSpec((RB, 1), lambda p, i: (i, 0)),
                  pl.BlockSpec((1, H), lambda p, i: (0, 0)),
                  pl.BlockSpec((1, H), lambda p, i: (0, 0)),
                  pl.BlockSpec((1, H), lambda p, i: (0, 0)),
                  pl.BlockSpec((H, H), lambda p, i: (0, 0)),
                  pl.BlockSpec((1, H), lambda p, i: (0, 0))],
        out_specs=pl.BlockSpec((RB, H), lambda p, i: (p * i, 0)),
        out_shape=jax.ShapeDtypeStruct((N, H), jnp.float32),
        scratch_shapes=[pltpu.VMEM((N, H), jnp.float32),
                        pltpu.VMEM((2, H), jnp.float32)],
    )(agg, hp, dinv, b, g, be, W, bl)


def kernel(x, edge_index, W1, b1, g1, be1, W2, b2, g2, be2,
           W3, b3, g3, be3, Wl, bl):
    _sc_degree, _sc_aggregate = _sc_kernels()
    ei3 = edge_index.reshape(2, NT * NCHT, CH)

    degs = _sc_degree(edge_index[1])
    degT = degs.T

    b1r, g1r, be1r = b1.reshape(1, H), g1.reshape(1, H), be1.reshape(1, H)
    b2r, g2r, be2r = b2.reshape(1, H), g2.reshape(1, H), be2.reshape(1, H)
    b3r, g3r, be3r = b3.reshape(1, H), g3.reshape(1, H), be3.reshape(1, H)
    blr = bl.reshape(1, H)

    hp, dinv = _tc_first(x, W1, degT)

    agg = _sc_aggregate(hp, ei3)
    hp = _tc_postmm(agg, hp, dinv, b1r, g1r, be1r, W2)

    agg = _sc_aggregate(hp, ei3)
    hp = _tc_postmm(agg, hp, dinv, b2r, g2r, be2r, W3)

    agg = _sc_aggregate(hp, ei3)
    return _tc_postfinal(agg, hp, dinv, b3r, g3r, be3r, Wl, blr)


# speedup vs baseline: 1.0355x; 1.0355x over previous
import functools

import jax
import jax.numpy as jnp
from jax import lax
from jax.experimental import pallas as pl
from jax.experimental.pallas import tpu as pltpu
from jax.experimental.pallas import tpu_sc as plsc

N = 10000
E = 320000
H = 128
NC = 2
NS = 16
NT = NC * NS
NP = 10240
EPT = E // NT
CH = 125
NCHT = EPT // CH
NPS = N // NS
RB = 400
GRID = N // RB


@functools.cache
def _sc_kernels():
    mesh = plsc.VectorSubcoreMesh(core_axis_name="c", subcore_axis_name="s",
                                  num_cores=NC, num_subcores=NS)

    @functools.partial(
        pl.kernel,
        out_type=jax.ShapeDtypeStruct((NT, NP), jnp.float32),
        mesh=mesh,
        compiler_params=pltpu.CompilerParams(needs_layout_passes=False),
        scratch_types=[
            pltpu.VMEM((EPT,), jnp.int32),
            pltpu.VMEM((NP,), jnp.float32),
        ],
    )
    def _sc_degree(dst_hbm, out_hbm, dst_v, hist_v):
        cid = lax.axis_index("c")
        sid = lax.axis_index("s")
        wid = cid * NS + sid
        zero = jnp.zeros((16,), jnp.float32)

        def zbody(i, carry):
            hist_v[pl.ds(i * 16, 16)] = zero
            return carry

        lax.fori_loop(0, NP // 16, zbody, 0)
        pltpu.sync_copy(dst_hbm.at[pl.ds(wid * EPT, EPT)], dst_v)
        ones = jnp.ones((16,), jnp.float32)

        def body(i, carry):
            idx = dst_v[pl.ds(i * 16, 16)]
            plsc.addupdate_scatter(hist_v, [idx], ones)
            return carry

        lax.fori_loop(0, EPT // 16, body, 0)
        pltpu.sync_copy(hist_v, out_hbm.at[wid])

    @functools.partial(
        pl.kernel,
        out_type=jax.ShapeDtypeStruct((NC, N, H), jnp.float32),
        mesh=mesh,
        compiler_params=pltpu.CompilerParams(needs_layout_passes=False,
                                             use_tc_tiling_on_sc=False),
        scratch_types=[
            pltpu.VMEM((NCHT, CH), jnp.int32),
            pltpu.VMEM((NCHT // 2, CH), jnp.int32),
            pltpu.VMEM((CH, H), jnp.float32),
            pltpu.VMEM((CH, H), jnp.float32),
            pltpu.VMEM_SHARED((N, H), jnp.float32),
            pltpu.SemaphoreType.DMA,
            pltpu.SemaphoreType.DMA,
        ],
    )
    def _sc_aggregate(hp_hbm, ei_hbm, out_hbm,
                      sidx_v, didx_v, rows0_v, rows1_v, acc_sh,
                      semA, semB):
        cid = lax.axis_index("c")
        sid = lax.axis_index("s")
        wid = cid * NS + sid
        zero = jnp.zeros((16,), jnp.float32)

        def zbody(i, carry):
            rows0_v[i >> 3, pl.ds((i & 7) * 16, 16)] = zero
            return carry

        lax.fori_loop(0, CH * (H // 16), zbody, 0)
        base = sid * NPS
        for j in range(NPS // CH):
            pltpu.sync_copy(rows0_v, acc_sh.at[pl.ds(base + j * CH, CH)])
        plsc.subcore_barrier()

        pltpu.sync_copy(ei_hbm.at[0, pl.ds(wid * NCHT, NCHT)], sidx_v)
        pltpu.sync_copy(ei_hbm.at[1, pl.ds(wid * NCHT, NCHT // 2)], didx_v)

        def _wait(buf, sem):
            pltpu.make_async_copy(hp_hbm.at[pl.ds(0, CH)], buf, sem).wait()

        pltpu.async_copy(hp_hbm.at[sidx_v.at[0]], rows0_v, semA)

        def body(i, carry):
            j = 2 * i

            @pl.when(i == NCHT // 4)
            def _():
                pltpu.sync_copy(
                    ei_hbm.at[1, pl.ds(wid * NCHT + NCHT // 2, NCHT // 2)],
                    didx_v)

            jr = jnp.where(j >= NCHT // 2, j - NCHT // 2, j)
            pltpu.async_copy(hp_hbm.at[sidx_v.at[j + 1]], rows1_v, semB)
            _wait(rows0_v, semA)
            pltpu.sync_copy(rows0_v, acc_sh.at[didx_v.at[jr]], add=True)

            @pl.when(j + 2 < NCHT)
            def _():
                pltpu.async_copy(hp_hbm.at[sidx_v.at[j + 2]], rows0_v, semA)

            _wait(rows1_v, semB)
            pltpu.sync_copy(rows1_v, acc_sh.at[didx_v.at[jr + 1]], add=True)
            return carry

        lax.fori_loop(0, NCHT // 2, body, 0)
        plsc.subcore_barrier()
        for j in range(NPS // CH):
            pltpu.sync_copy(acc_sh.at[pl.ds(base + j * CH, CH)],
                            out_hbm.at[cid, pl.ds(base + j * CH, CH)])

    return _sc_degree, _sc_aggregate


def _dot(a, b):
    return jnp.dot(a, b, preferred_element_type=jnp.float32,
                   precision=jax.lax.Precision.HIGHEST)


def _tc_first_body(x_ref, w_ref, degT_ref, hp_ref, dinv_ref):
    deg = jnp.sum(degT_ref[...], axis=1, keepdims=True) + 1.0
    dinv = lax.rsqrt(deg)
    h = _dot(x_ref[...], w_ref[...])
    hp_ref[...] = h * dinv
    dinv_ref[...] = dinv


def _tc_first(x, W, degT):
    return pl.pallas_call(
        _tc_first_body,
        grid=(GRID,),
        in_specs=[pl.BlockSpec((RB, H), lambda i: (i, 0)),
                  pl.BlockSpec((H, H), lambda i: (0, 0)),
                  pl.BlockSpec((RB, NT), lambda i: (i, 0))],
        out_specs=[pl.BlockSpec((RB, H), lambda i: (i, 0)),
                   pl.BlockSpec((RB, 1), lambda i: (i, 0))],
        out_shape=[jax.ShapeDtypeStruct((N, H), jnp.float32),
                   jax.ShapeDtypeStruct((N, 1), jnp.float32)],
    )(x, W, degT)


def _bn_coeffs(stats, g, be):
    m = stats[0:1, :] * (1.0 / N)
    v = stats[1:2, :] * (1.0 / N) - m * m
    a = g * lax.rsqrt(v + 1e-5)
    return a, be - m * a


def _post_phase(agg_ref, hp_ref, dinv_ref, b_ref, zs_ref, stats_ref, i):
    s = agg_ref[0] + agg_ref[1] + hp_ref[...]
    z = jnp.maximum(s * dinv_ref[...] + b_ref[...], 0.0)
    zs_ref[pl.ds(i * RB, RB), :] = z
    new = jnp.concatenate([jnp.sum(z, axis=0, keepdims=True),
                           jnp.sum(z * z, axis=0, keepdims=True)], axis=0)

    @pl.when(i == 0)
    def _():
        stats_ref[...] = new

    @pl.when(i > 0)
    def _():
        stats_ref[...] += new


def _tc_postmm_body(agg_ref, hp_ref, dinv_ref, b_ref, g_ref, be_ref, w_ref,
                    out_ref, zs_ref, stats_ref):
    p = pl.program_id(0)
    i = pl.program_id(1)

    @pl.when(p == 0)
    def _():
        _post_phase(agg_ref, hp_ref, dinv_ref, b_ref, zs_ref, stats_ref, i)

    @pl.when(p == 1)
    def _():
        a, c = _bn_coeffs(stats_ref[...], g_ref[...], be_ref[...])
        z = zs_ref[pl.ds(i * RB, RB), :]
        h = _dot(z * a, w_ref[...]) + _dot(c, w_ref[...])
        out_ref[...] = h * dinv_ref[...]


def _tc_postmm(agg, hp, dinv, b, g, be, W):
    return pl.pallas_call(
        _tc_postmm_body,
        grid=(2, GRID),
        in_specs=[pl.BlockSpec((NC, RB, H), lambda p, i: (0, i, 0)),
                  pl.BlockSpec((RB, H), lambda p, i: (i, 0)),
                  pl.BlockSpec((RB, 1), lambda p, i: (i, 0)),
                  pl.BlockSpec((1, H), lambda p, i: (0, 0)),
                  pl.BlockSpec((1, H), lambda p, i: (0, 0)),
                  pl.BlockSpec((1, H), lambda p, i: (0, 0)),
                  pl.BlockSpec((H, H), lambda p, i: (0, 0))],
        out_specs=pl.BlockSpec((RB, H), lambda p, i: (p * i, 0)),
        out_shape=jax.ShapeDtypeStruct((N, H), jnp.float32),
        scratch_shapes=[pltpu.VMEM((N, H), jnp.float32),
                        pltpu.VMEM((2, H), jnp.float32)],
    )(agg, hp, dinv, b, g, be, W)


def _tc_postfinal_body(agg_ref, hp_ref, dinv_ref, b_ref, g_ref, be_ref,
                       w_ref, bl_ref, out_ref, zs_ref, stats_ref):
    p = pl.program_id(0)
    i = pl.program_id(1)

    @pl.when(p == 0)
    def _():
        _post_phase(agg_ref, hp_ref, dinv_ref, b_ref, zs_ref, stats_ref, i)

    @pl.when(p == 1)
    def _():
        a, c = _bn_coeffs(stats_ref[...], g_ref[...], be_ref[...])
        z = zs_ref[pl.ds(i * RB, RB), :]
        t = _dot(z * a, w_ref[...]) + _dot(c, w_ref[...]) + bl_ref[...]
        r = jnp.maximum(t, 0.0)
        e = jnp.exp(r - jnp.max(r, axis=1, keepdims=True))
        out_ref[...] = e / jnp.sum(e, axis=1, keepdims=True)


def _tc_postfinal(agg, hp, dinv, b, g, be, W, bl):
    return pl.pallas_call(
        _tc_postfinal_body,
        grid=(2, GRID),
        in_specs=[pl.BlockSpec((NC, RB, H), lambda p, i: (0, i, 0)),
                  pl.BlockSpec((RB, H), lambda p, i: (i, 0)),
                  pl.BlockSpec((RB, 1), lambda p, i: (i, 0)),
                  pl.BlockSpec((1, H), lambda p, i: (0, 0)),
                  pl.BlockSpec((1, H), lambda p, i: (0, 0)),
                  pl.BlockSpec((1, H), lambda p, i: (0, 0)),
                  pl.BlockSpec((H, H), lambda p, i: (0, 0)),
                  pl.BlockSpec((1, H), lambda p, i: (0, 0))],
        out_specs=pl.BlockSpec((RB, H), lambda p, i: (p * i, 0)),
        out_shape=jax.ShapeDtypeStruct((N, H), jnp.float32),
        scratch_shapes=[pltpu.VMEM((N, H), jnp.float32),
                        pltpu.VMEM((2, H), jnp.float32)],
    )(agg, hp, dinv, b, g, be, W, bl)


def kernel(x, edge_index, W1, b1, g1, be1, W2, b2, g2, be2,
           W3, b3, g3, be3, Wl, bl):
    _sc_degree, _sc_aggregate = _sc_kernels()
    ei3 = edge_index.reshape(2, NT * NCHT, CH)

    degs = _sc_degree(edge_index[1])
    degT = degs.T

    b1r, g1r, be1r = b1.reshape(1, H), g1.reshape(1, H), be1.reshape(1, H)
    b2r, g2r, be2r = b2.reshape(1, H), g2.reshape(1, H), be2.reshape(1, H)
    b3r, g3r, be3r = b3.reshape(1, H), g3.reshape(1, H), be3.reshape(1, H)
    blr = bl.reshape(1, H)

    hp, dinv = _tc_first(x, W1, degT)

    agg = _sc_aggregate(hp, ei3)
    hp = _tc_postmm(agg, hp, dinv, b1r, g1r, be1r, W2)

    agg = _sc_aggregate(hp, ei3)
    hp = _tc_postmm(agg, hp, dinv, b2r, g2r, be2r, W3)

    agg = _sc_aggregate(hp, ei3)
    return _tc_postfinal(agg, hp, dinv, b3r, g3r, be3r, Wl, blr)


# speedup vs baseline: 1.0550x; 1.0188x over previous
import functools

import jax
import jax.numpy as jnp
from jax import lax
from jax.experimental import pallas as pl
from jax.experimental.pallas import tpu as pltpu
from jax.experimental.pallas import tpu_sc as plsc

N = 10000
E = 320000
H = 128
NC = 2
NS = 16
NT = NC * NS
NP = 10240
EPT = E // NT
CH = 125
NCHT = EPT // CH
NPS = N // NS
RB = 400
GRID = N // RB


@functools.cache
def _sc_kernels():
    mesh = plsc.VectorSubcoreMesh(core_axis_name="c", subcore_axis_name="s",
                                  num_cores=NC, num_subcores=NS)

    @functools.partial(
        pl.kernel,
        out_type=jax.ShapeDtypeStruct((NT, NP), jnp.float32),
        mesh=mesh,
        compiler_params=pltpu.CompilerParams(needs_layout_passes=False),
        scratch_types=[
            pltpu.VMEM((EPT,), jnp.int32),
            pltpu.VMEM((NP,), jnp.float32),
        ],
    )
    def _sc_degree(dst_hbm, out_hbm, dst_v, hist_v):
        cid = lax.axis_index("c")
        sid = lax.axis_index("s")
        wid = cid * NS + sid
        zero = jnp.zeros((16,), jnp.float32)

        def zbody(i, carry):
            hist_v[pl.ds(i * 16, 16)] = zero
            return carry

        lax.fori_loop(0, NP // 16, zbody, 0)
        pltpu.sync_copy(dst_hbm.at[pl.ds(wid * EPT, EPT)], dst_v)
        ones = jnp.ones((16,), jnp.float32)

        def body(i, carry):
            idx = dst_v[pl.ds(i * 16, 16)]
            plsc.addupdate_scatter(hist_v, [idx], ones)
            return carry

        lax.fori_loop(0, EPT // 16, body, 0)
        pltpu.sync_copy(hist_v, out_hbm.at[wid])

    @functools.partial(
        pl.kernel,
        out_type=jax.ShapeDtypeStruct((NC, N, H), jnp.float32),
        mesh=mesh,
        compiler_params=pltpu.CompilerParams(needs_layout_passes=False,
                                             use_tc_tiling_on_sc=False),
        scratch_types=[
            pltpu.VMEM((NCHT, CH), jnp.int32),
            pltpu.VMEM((NCHT // 2, CH), jnp.int32),
            pltpu.VMEM((CH, H), jnp.float32),
            pltpu.VMEM((CH, H), jnp.float32),
            pltpu.VMEM_SHARED((N, H), jnp.float32),
            pltpu.SemaphoreType.DMA,
            pltpu.SemaphoreType.DMA,
        ],
    )
    def _sc_aggregate(hp_hbm, ei_hbm, out_hbm,
                      sidx_v, didx_v, rows0_v, rows1_v, acc_sh,
                      semA, semB):
        cid = lax.axis_index("c")
        sid = lax.axis_index("s")
        wid = cid * NS + sid
        zero = jnp.zeros((16,), jnp.float32)

        def zbody(i, carry):
            rows0_v[i >> 3, pl.ds((i & 7) * 16, 16)] = zero
            return carry

        lax.fori_loop(0, CH * (H // 16), zbody, 0)
        base = sid * NPS
        for j in range(NPS // CH):
            pltpu.sync_copy(rows0_v, acc_sh.at[pl.ds(base + j * CH, CH)])
        plsc.subcore_barrier()

        pltpu.sync_copy(ei_hbm.at[0, pl.ds(wid * NCHT, NCHT)], sidx_v)
        pltpu.sync_copy(ei_hbm.at[1, pl.ds(wid * NCHT, NCHT // 2)], didx_v)

        def _wait(buf, sem):
            pltpu.make_async_copy(hp_hbm.at[pl.ds(0, CH)], buf, sem).wait()

        pltpu.async_copy(hp_hbm.at[sidx_v.at[0]], rows0_v, semA)

        def body(i, carry):
            j = 2 * i

            @pl.when(i == NCHT // 4)
            def _():
                pltpu.sync_copy(
                    ei_hbm.at[1, pl.ds(wid * NCHT + NCHT // 2, NCHT // 2)],
                    didx_v)

            jr = jnp.where(j >= NCHT // 2, j - NCHT // 2, j)
            pltpu.async_copy(hp_hbm.at[sidx_v.at[j + 1]], rows1_v, semB)
            _wait(rows0_v, semA)
            pltpu.sync_copy(rows0_v, acc_sh.at[didx_v.at[jr]], add=True)

            @pl.when(j + 2 < NCHT)
            def _():
                pltpu.async_copy(hp_hbm.at[sidx_v.at[j + 2]], rows0_v, semA)

            _wait(rows1_v, semB)
            pltpu.sync_copy(rows1_v, acc_sh.at[didx_v.at[jr + 1]], add=True)
            return carry

        lax.fori_loop(0, NCHT // 2, body, 0)
        plsc.subcore_barrier()
        for j in range(NPS // CH):
            pltpu.sync_copy(acc_sh.at[pl.ds(base + j * CH, CH)],
                            out_hbm.at[cid, pl.ds(base + j * CH, CH)])

    return _sc_degree, _sc_aggregate


def _dot(a, b):
    return jnp.dot(a, b, preferred_element_type=jnp.float32,
                   precision=jax.lax.Precision.HIGHEST)


def _tc_first_body(x_ref, w_ref, degT_ref, hp_ref, dinv_ref):
    deg = jnp.sum(degT_ref[...], axis=1, keepdims=True) + 1.0
    dinv = lax.rsqrt(deg)
    h = _dot(x_ref[...], w_ref[...])
    hp_ref[...] = h * dinv
    dinv_ref[...] = dinv


def _tc_first(x, W, degT):
    return pl.pallas_call(
        _tc_first_body,
        grid=(GRID,),
        in_specs=[pl.BlockSpec((RB, H), lambda i: (i, 0)),
                  pl.BlockSpec((H, H), lambda i: (0, 0)),
                  pl.BlockSpec((RB, NT), lambda i: (i, 0))],
        out_specs=[pl.BlockSpec((RB, H), lambda i: (i, 0)),
                   pl.BlockSpec((RB, 1), lambda i: (i, 0))],
        out_shape=[jax.ShapeDtypeStruct((N, H), jnp.float32),
                   jax.ShapeDtypeStruct((N, 1), jnp.float32)],
    )(x, W, degT)


def _bn_coeffs(stats, g, be):
    m = stats[0:1, :] * (1.0 / N)
    v = stats[1:2, :] * (1.0 / N) - m * m
    a = g * lax.rsqrt(v + 1e-5)
    return a, be - m * a


def _post_phase(agg_ref, hp_ref, dinv_ref, b_ref, zs_ref, stats_ref, i):
    s = agg_ref[0] + agg_ref[1] + hp_ref[...]
    z = jnp.maximum(s * dinv_ref[...] + b_ref[...], 0.0)
    zs_ref[pl.ds(i * RB, RB), :] = z
    new = jnp.concatenate([jnp.sum(z, axis=0, keepdims=True),
                           jnp.sum(z * z, axis=0, keepdims=True)], axis=0)

    @pl.when(i == 0)
    def _():
        stats_ref[...] = new

    @pl.when(i > 0)
    def _():
        stats_ref[...] += new


def _tc_postmm_body(agg_ref, hp_ref, dinv_ref, b_ref, g_ref, be_ref, w_ref,
                    out_ref, zs_ref, stats_ref):
    p = pl.program_id(0)
    i = pl.program_id(1)

    @pl.when(p == 0)
    def _():
        _post_phase(agg_ref, hp_ref, dinv_ref, b_ref, zs_ref, stats_ref, i)

    @pl.when(p == 1)
    def _():
        a, c = _bn_coeffs(stats_ref[...], g_ref[...], be_ref[...])
        z = zs_ref[pl.ds(i * RB, RB), :]
        h = _dot(z * a, w_ref[...]) + _dot(c, w_ref[...])
        out_ref[...] = h * dinv_ref[...]


def _tc_postmm(agg, hp, dinv, b, g, be, W):
    return pl.pallas_call(
        _tc_postmm_body,
        grid=(2, GRID),
        in_specs=[pl.BlockSpec((NC, RB, H), lambda p, i: (0, (1 - p) * i, 0)),
                  pl.BlockSpec((RB, H), lambda p, i: ((1 - p) * i, 0)),
                  pl.BlockSpec((RB, 1), lambda p, i: (i, 0)),
                  pl.BlockSpec((1, H), lambda p, i: (0, 0)),
                  pl.BlockSpec((1, H), lambda p, i: (0, 0)),
                  pl.BlockSpec((1, H), lambda p, i: (0, 0)),
                  pl.BlockSpec((H, H), lambda p, i: (0, 0))],
        out_specs=pl.BlockSpec((RB, H), lambda p, i: (p * i, 0)),
        out_shape=jax.ShapeDtypeStruct((N, H), jnp.float32),
        scratch_shapes=[pltpu.VMEM((N, H), jnp.float32),
                        pltpu.VMEM((2, H), jnp.float32)],
    )(agg, hp, dinv, b, g, be, W)


def _tc_postfinal_body(agg_ref, hp_ref, dinv_ref, b_ref, g_ref, be_ref,
                       w_ref, bl_ref, out_ref, zs_ref, stats_ref):
    p = pl.program_id(0)
    i = pl.program_id(1)

    @pl.when(p == 0)
    def _():
        _post_phase(agg_ref, hp_ref, dinv_ref, b_ref, zs_ref, stats_ref, i)

    @pl.when(p == 1)
    def _():
        a, c = _bn_coeffs(stats_ref[...], g_ref[...], be_ref[...])
        z = zs_ref[pl.ds(i * RB, RB), :]
        t = _dot(z * a, w_ref[...]) + _dot(c, w_ref[...]) + bl_ref[...]
        r = jnp.maximum(t, 0.0)
        e = jnp.exp(r - jnp.max(r, axis=1, keepdims=True))
        out_ref[...] = e / jnp.sum(e, axis=1, keepdims=True)


def _tc_postfinal(agg, hp, dinv, b, g, be, W, bl):
    return pl.pallas_call(
        _tc_postfinal_body,
        grid=(2, GRID),
        in_specs=[pl.BlockSpec((NC, RB, H), lambda p, i: (0, (1 - p) * i, 0)),
                  pl.BlockSpec((RB, H), lambda p, i: ((1 - p) * i, 0)),
                  pl.BlockSpec((RB, 1), lambda p, i: (i, 0)),
                  pl.BlockSpec((1, H), lambda p, i: (0, 0)),
                  pl.BlockSpec((1, H), lambda p, i: (0, 0)),
                  pl.BlockSpec((1, H), lambda p, i: (0, 0)),
                  pl.BlockSpec((H, H), lambda p, i: (0, 0)),
                  pl.BlockSpec((1, H), lambda p, i: (0, 0))],
        out_specs=pl.BlockSpec((RB, H), lambda p, i: (p * i, 0)),
        out_shape=jax.ShapeDtypeStruct((N, H), jnp.float32),
        scratch_shapes=[pltpu.VMEM((N, H), jnp.float32),
                        pltpu.VMEM((2, H), jnp.float32)],
    )(agg, hp, dinv, b, g, be, W, bl)


def kernel(x, edge_index, W1, b1, g1, be1, W2, b2, g2, be2,
           W3, b3, g3, be3, Wl, bl):
    _sc_degree, _sc_aggregate = _sc_kernels()
    ei3 = edge_index.reshape(2, NT * NCHT, CH)

    degs = _sc_degree(edge_index[1])
    degT = degs.T

    b1r, g1r, be1r = b1.reshape(1, H), g1.reshape(1, H), be1.reshape(1, H)
    b2r, g2r, be2r = b2.reshape(1, H), g2.reshape(1, H), be2.reshape(1, H)
    b3r, g3r, be3r = b3.reshape(1, H), g3.reshape(1, H), be3.reshape(1, H)
    blr = bl.reshape(1, H)

    hp, dinv = _tc_first(x, W1, degT)

    agg = _sc_aggregate(hp, ei3)
    hp = _tc_postmm(agg, hp, dinv, b1r, g1r, be1r, W2)

    agg = _sc_aggregate(hp, ei3)
    hp = _tc_postmm(agg, hp, dinv, b2r, g2r, be2r, W3)

    agg = _sc_aggregate(hp, ei3)
    return _tc_postfinal(agg, hp, dinv, b3r, g3r, be3r, Wl, blr)
